# Initial kernel scaffold; baseline (speedup 1.0000x reference)
import jax
import jax.numpy as jnp
from jax import lax
from jax.experimental import pallas as pl
from jax.experimental.pallas import tpu as pltpu
from jax.experimental.pallas import tpu_sc as plsc

B, H, W, C = 16, 384, 384, 96
PIX = B * H * W
NC, NS, L = 2, 16, 16
NW = NC * NS
PPW = PIX // NW
N = 96
CHUNKS = PPW // N
G = N // L
Wf = float(W)
Hf = float(H)


def _gen_and_fire(c, pix0, base_v, xf_hbm, yf_hbm, im_hbm,
                  cx, cy, idxs, ws, gbufs, sem):
    off = pix0 + c * N
    pltpu.sync_copy(xf_hbm.at[pl.ds(off, N)], cx)
    pltpu.sync_copy(yf_hbm.at[pl.ds(off, N)], cy)
    idxA, idxB, idxC, idxD = idxs
    wA, wB, wC, wD = ws
    for g in range(G):
        s = pl.ds(g * L, L)
        Xf = cx[s]
        Yf = cy[s]
        Xp = (Xf + 1.0) / 2.0 * Wf
        Yp = (Yf + 1.0) / 2.0 * Hf
        x0t = Xp.astype(jnp.int32)
        x0 = jnp.where(x0t.astype(jnp.float32) > Xp, x0t - 1, x0t)
        y0t = Yp.astype(jnp.int32)
        y0 = jnp.where(y0t.astype(jnp.float32) > Yp, y0t - 1, y0t)
        x1 = x0 + 1
        y1 = y0 + 1
        x0c = jnp.minimum(jnp.maximum(x0, 0), W - 1)
        x1c = jnp.minimum(jnp.maximum(x1, 0), W - 1)
        y0c = jnp.minimum(jnp.maximum(y0, 0), H - 1)
        y1c = jnp.minimum(jnp.maximum(y1, 0), H - 1)
        x0f = x0c.astype(jnp.float32)
        x1f = x1c.astype(jnp.float32)
        y0f = y0c.astype(jnp.float32)
        y1f = y1c.astype(jnp.float32)
        wa = (x1f - Xp) * (y1f - Yp)
        wb = (x1f - Xp) * (Yp - y0f)
        wc = (Xp - x0f) * (y1f - Yp)
        wd = (Xp - x0f) * (Yp - y0f)
        rA = base_v + y0c * W + x0c
        rB = base_v + y1c * W + x0c
        rC = base_v + y0c * W + x1c
        rD = base_v + y1c * W + x1c
        idxA[s] = rA
        idxB[s] = rB
        idxC[s] = rC
        idxD[s] = rD
        wA[s] = wa
        wB[s] = wb
        wC[s] = wc
        wD[s] = wd
    gA, gB, gC, gD = gbufs
    pltpu.async_copy(im_hbm.at[idxA], gA, sem)
    pltpu.async_copy(im_hbm.at[idxB], gB, sem)
    pltpu.async_copy(im_hbm.at[idxC], gC, sem)
    pltpu.async_copy(im_hbm.at[idxD], gD, sem)


def _drain_gathers(im_hbm, idxs, gbufs, sem):
    for ix, gb in zip(idxs, gbufs):
        pltpu.make_async_copy(im_hbm.at[ix], gb, sem).wait()


def _combine(ws, gbufs, ob):
    wA, wB, wC, wD = ws
    gA, gB, gC, gD = gbufs

    def gbody(g, carry):
        s = pl.ds(g * L, L)
        wa16 = wA[s]
        wb16 = wB[s]
        wc16 = wC[s]
        wd16 = wD[s]
        for l in range(L):
            i = g * L + l
            sel = jnp.full((L,), l, jnp.int32)
            wal = jnp.take(wa16, sel, axis=0, mode="promise_in_bounds")
            wbl = jnp.take(wb16, sel, axis=0, mode="promise_in_bounds")
            wcl = jnp.take(wc16, sel, axis=0, mode="promise_in_bounds")
            wdl = jnp.take(wd16, sel, axis=0, mode="promise_in_bounds")
            for j in range(C // L):
                cs = pl.ds(j * L, L)
                o = gA[i, cs] * wal + gB[i, cs] * wbl
                o = o + gC[i, cs] * wcl
                o = o + gD[i, cs] * wdl
                ob[i, cs] = o
        return carry

    lax.fori_loop(0, G, gbody, 0)


def _body(im_hbm, xf_hbm, yf_hbm, out_hbm,
          cx0, cy0, cx1, cy1,
          iA0, iB0, iC0, iD0, iA1, iB1, iC1, iD1,
          wA0, wB0, wC0, wD0, wA1, wB1, wC1, wD1,
          gA0, gB0, gC0, gD0, gA1, gB1, gC1, gD1,
          ob0, ob1,
          sem_g0, sem_g1, sem_o0, sem_o1):
    wid = lax.axis_index("c") * NS + lax.axis_index("s")
    pix0 = wid * PPW
    img_base = (wid // 2) * (H * W)
    base_v = jnp.full((L,), img_base, jnp.int32)

    cxs = (cx0, cx1)
    cys = (cy0, cy1)
    idxs = ((iA0, iB0, iC0, iD0), (iA1, iB1, iC1, iD1))
    ws = ((wA0, wB0, wC0, wD0), (wA1, wB1, wC1, wD1))
    gbufs = ((gA0, gB0, gC0, gD0), (gA1, gB1, gC1, gD1))
    obs = (ob0, ob1)
    sem_gs = (sem_g0, sem_g1)
    sem_os = (sem_o0, sem_o1)

    _gen_and_fire(0, pix0, base_v, xf_hbm, yf_hbm, im_hbm,
                  cxs[0], cys[0], idxs[0], ws[0], gbufs[0], sem_gs[0])

    def outer(i2, carry):
        for d in (0, 1):
            c = i2 * 2 + d
            nd = 1 - d

            @pl.when(c + 1 < CHUNKS)
            def _():
                _gen_and_fire(c + 1, pix0, base_v, xf_hbm, yf_hbm, im_hbm,
                              cxs[nd], cys[nd], idxs[nd], ws[nd],
                              gbufs[nd], sem_gs[nd])

            _drain_gathers(im_hbm, idxs[d], gbufs[d], sem_gs[d])

            @pl.when(c >= 2)
            def _():
                prev = pix0 + (c - 2) * N
                pltpu.make_async_copy(
                    obs[d], out_hbm.at[pl.ds(prev, N)], sem_os[d]).wait()

            _combine(ws[d], gbufs[d], obs[d])
            cur = pix0 + c * N
            pltpu.async_copy(obs[d], out_hbm.at[pl.ds(cur, N)], sem_os[d])
        return carry

    lax.fori_loop(0, CHUNKS // 2, outer, 0)

    for d in (0, 1):
        last = pix0 + (CHUNKS - 2 + d) * N
        pltpu.make_async_copy(
            obs[d], out_hbm.at[pl.ds(last, N)], sem_os[d]).wait()


@jax.jit
def kernel(im, thetas):
    X, Y = jnp.meshgrid(jnp.linspace(-1.0, 1.0, W), jnp.linspace(-1.0, 1.0, H))
    flat_coords = jnp.concatenate(
        [X.reshape(1, -1), Y.reshape(1, -1),
         jnp.ones((1, H * W), dtype=jnp.float32)], axis=0)
    th = thetas.reshape(-1, 2, 3)
    new_flat = jnp.matmul(th, jnp.broadcast_to(flat_coords[None, :, :],
                                               (B, 3, H * W)))
    Xf = new_flat[:, 0, :].reshape(-1)
    Yf = new_flat[:, 1, :].reshape(-1)
    im_flat = im.reshape(-1, C)

    mesh = plsc.VectorSubcoreMesh(core_axis_name="c", subcore_axis_name="s",
                                  num_cores=NC, num_subcores=NS)
    scratch = (
        [pltpu.VMEM((N,), jnp.float32) for _ in range(4)]
        + [pltpu.VMEM((N,), jnp.int32) for _ in range(8)]
        + [pltpu.VMEM((N,), jnp.float32) for _ in range(8)]
        + [pltpu.VMEM((N, C), jnp.float32) for _ in range(8)]
        + [pltpu.VMEM((N, C), jnp.float32) for _ in range(2)]
        + [pltpu.SemaphoreType.DMA for _ in range(4)]
    )
    out_flat = pl.kernel(
        _body,
        out_type=jax.ShapeDtypeStruct((PIX, C), jnp.float32),
        mesh=mesh,
        scratch_types=scratch,
    )(im_flat, Xf, Yf)
    return out_flat.reshape(B, H, W, C)


# speedup vs baseline: 1.4983x; 1.4983x over previous
import jax
import jax.numpy as jnp
from jax import lax
from jax.experimental import pallas as pl
from jax.experimental.pallas import tpu as pltpu
from jax.experimental.pallas import tpu_sc as plsc

B, H, W, C = 16, 384, 384, 96
PIX = B * H * W
NC, NS, L = 2, 16, 16
NW = NC * NS
PPW = PIX // NW
N = 96
CHUNKS = PPW // N
G = N // L
CB = 8
NCB = N * CB
Wf = float(W)
Hf = float(H)

_DNUMS = lax.GatherDimensionNumbers(
    offset_dims=(), collapsed_slice_dims=(0,), start_index_map=(0,))


def _splat(v, l):
    idx = jnp.full((L, 1), l, jnp.int32)
    return lax.gather(v, idx, _DNUMS, (1,),
                      mode=lax.GatherScatterMode.PROMISE_IN_BOUNDS)


def _gen_and_fire(c, pix0, base_v, xf_hbm, yf_hbm, nbr_hbm,
                  cbx, cby, idx, ws, gbuf, sem):
    @pl.when(lax.rem(c, CB) == 0)
    def _():
        off = pix0 + c * N
        pltpu.sync_copy(xf_hbm.at[pl.ds(off, NCB)], cbx)
        pltpu.sync_copy(yf_hbm.at[pl.ds(off, NCB)], cby)

    pos = lax.rem(c, CB) * N
    wA, wB, wC, wD = ws
    for g in range(G):
        s = pl.ds(g * L, L)
        Xf = cbx[pl.ds(pos + g * L, L)]
        Yf = cby[pl.ds(pos + g * L, L)]
        Xp = (Xf + 1.0) / 2.0 * Wf
        Yp = (Yf + 1.0) / 2.0 * Hf
        x0t = Xp.astype(jnp.int32)
        x0 = jnp.where(x0t.astype(jnp.float32) > Xp, x0t - 1, x0t)
        y0t = Yp.astype(jnp.int32)
        y0 = jnp.where(y0t.astype(jnp.float32) > Yp, y0t - 1, y0t)
        m = ((Xp >= 0.0) & (Xp < Wf - 1.0)
             & (Yp >= 0.0) & (Yp < Hf - 1.0))
        x0c = jnp.minimum(jnp.maximum(x0, 0), W - 2)
        y0c = jnp.minimum(jnp.maximum(y0, 0), H - 2)
        x0f = x0c.astype(jnp.float32)
        y0f = y0c.astype(jnp.float32)
        x1f = x0f + 1.0
        y1f = y0f + 1.0
        zero = jnp.zeros((L,), jnp.float32)
        wa = jnp.where(m, (x1f - Xp) * (y1f - Yp), zero)
        wb = jnp.where(m, (x1f - Xp) * (Yp - y0f), zero)
        wc = jnp.where(m, (Xp - x0f) * (y1f - Yp), zero)
        wd = jnp.where(m, (Xp - x0f) * (Yp - y0f), zero)
        idx[s] = base_v + y0c * W + x0c
        wA[s] = wa
        wB[s] = wb
        wC[s] = wc
        wD[s] = wd
    pltpu.async_copy(nbr_hbm.at[idx], gbuf, sem)


def _combine(ws, gbuf, ob):
    wA, wB, wC, wD = ws

    def gbody(g, carry):
        s = pl.ds(g * L, L)
        wa16 = wA[s]
        wb16 = wB[s]
        wc16 = wC[s]
        wd16 = wD[s]
        for l in range(L):
            i = g * L + l
            wal = _splat(wa16, l)
            wbl = _splat(wb16, l)
            wcl = _splat(wc16, l)
            wdl = _splat(wd16, l)
            for j in range(C // L):
                av = gbuf[i, pl.ds(j * L, L)]
                cv = gbuf[i, pl.ds(C + j * L, L)]
                bv = gbuf[i, pl.ds(2 * C + j * L, L)]
                dv = gbuf[i, pl.ds(3 * C + j * L, L)]
                o = av * wal + bv * wbl
                o = o + cv * wcl
                o = o + dv * wdl
                ob[pl.ds(i * C + j * L, L)] = o
        return carry

    lax.fori_loop(0, G, gbody, 0)


def _body(nbr_hbm, xf_hbm, yf_hbm, out_hbm,
          cbx, cby,
          idx0, idx1,
          wA0, wB0, wC0, wD0, wA1, wB1, wC1, wD1,
          g0, g1, ob0, ob1,
          sem_g0, sem_g1, sem_o0, sem_o1):
    wid = lax.axis_index("c") * NS + lax.axis_index("s")
    pix0 = wid * PPW
    img_base = (wid // 2) * (H * W)
    base_v = jnp.full((L,), img_base, jnp.int32)

    idxs = (idx0, idx1)
    ws = ((wA0, wB0, wC0, wD0), (wA1, wB1, wC1, wD1))
    gbufs = (g0, g1)
    obs = (ob0, ob1)
    sem_gs = (sem_g0, sem_g1)
    sem_os = (sem_o0, sem_o1)

    _gen_and_fire(0, pix0, base_v, xf_hbm, yf_hbm, nbr_hbm,
                  cbx, cby, idxs[0], ws[0], gbufs[0], sem_gs[0])

    def outer(i2, carry):
        for d in (0, 1):
            c = i2 * 2 + d
            nd = 1 - d

            @pl.when(c + 1 < CHUNKS)
            def _():
                _gen_and_fire(c + 1, pix0, base_v, xf_hbm, yf_hbm, nbr_hbm,
                              cbx, cby, idxs[nd], ws[nd], gbufs[nd],
                              sem_gs[nd])

            pltpu.make_async_copy(nbr_hbm.at[idxs[d]], gbufs[d],
                                  sem_gs[d]).wait()

            @pl.when(c >= 2)
            def _():
                prev = (pix0 + (c - 2) * N) * C
                pltpu.make_async_copy(
                    obs[d], out_hbm.at[pl.ds(prev, N * C)], sem_os[d]).wait()

            _combine(ws[d], gbufs[d], obs[d])
            cur = (pix0 + c * N) * C
            pltpu.async_copy(obs[d], out_hbm.at[pl.ds(cur, N * C)], sem_os[d])
        return carry

    lax.fori_loop(0, CHUNKS // 2, outer, 0)

    for d in (0, 1):
        last = (pix0 + (CHUNKS - 2 + d) * N) * C
        pltpu.make_async_copy(
            obs[d], out_hbm.at[pl.ds(last, N * C)], sem_os[d]).wait()


@jax.jit
def kernel(im, thetas):
    X, Y = jnp.meshgrid(jnp.linspace(-1.0, 1.0, W), jnp.linspace(-1.0, 1.0, H))
    flat_coords = jnp.concatenate(
        [X.reshape(1, -1), Y.reshape(1, -1),
         jnp.ones((1, H * W), dtype=jnp.float32)], axis=0)
    th = thetas.reshape(-1, 2, 3)
    new_flat = jnp.matmul(th, jnp.broadcast_to(flat_coords[None, :, :],
                                               (B, 3, H * W)))
    Xf = new_flat[:, 0, :].reshape(-1)
    Yf = new_flat[:, 1, :].reshape(-1)

    im_flat = im.reshape(-1, C)
    nbr = jnp.concatenate(
        [im_flat,
         jnp.roll(im_flat, -1, axis=0),
         jnp.roll(im_flat, -W, axis=0),
         jnp.roll(im_flat, -(W + 1), axis=0)], axis=1)

    mesh = plsc.VectorSubcoreMesh(core_axis_name="c", subcore_axis_name="s",
                                  num_cores=NC, num_subcores=NS)
    scratch = (
        [pltpu.VMEM((NCB,), jnp.float32) for _ in range(2)]
        + [pltpu.VMEM((N,), jnp.int32) for _ in range(2)]
        + [pltpu.VMEM((N,), jnp.float32) for _ in range(8)]
        + [pltpu.VMEM((N, 4 * C), jnp.float32) for _ in range(2)]
        + [pltpu.VMEM((N * C,), jnp.float32) for _ in range(2)]
        + [pltpu.SemaphoreType.DMA for _ in range(4)]
    )
    out_flat = pl.kernel(
        _body,
        out_type=jax.ShapeDtypeStruct((PIX * C,), jnp.float32),
        mesh=mesh,
        scratch_types=scratch,
    )(nbr, Xf, Yf)
    return out_flat.reshape(B, H, W, C)
